# baseline (speedup 1.0000x reference)
import jax
import jax.numpy as jnp
from jax import lax
from jax.experimental import pallas as pl
from jax.experimental.pallas import tpu as pltpu

BF16 = jnp.bfloat16
F32 = jnp.float32


def _up2x_rows(td, rows, w2, c):
    t = td.reshape(rows, w2, c)
    t = jnp.broadcast_to(t[:, :, None, :], (rows, w2, 2, c)).reshape(rows, 2 * w2, c)
    t = jnp.broadcast_to(t[:, None, :, :], (rows, 2, 2 * w2, c)).reshape(2 * rows, 2 * w2, c)
    return t.reshape(4 * rows * w2, c)


def _up2x_cols(row, w2, c):
    return jnp.broadcast_to(row[:, None, :], (w2, 2, c)).reshape(2 * w2, c)


def _conv3x3_acc(val, w3_ref, b3_row, bc_ref, bl_ref, br_ref, acc_ref, th, w, c):
    m2 = (th + 2) * w
    bc_ref[...] = val
    xix = lax.broadcasted_iota(jnp.int32, (m2, 1), 0) % w
    zrow = jnp.zeros((1, c), BF16)
    vl = jnp.concatenate([val[1:], zrow], axis=0)
    bl_ref[...] = jnp.where(xix == w - 1, zrow, vl)
    vr = jnp.concatenate([zrow, val[:-1]], axis=0)
    br_ref[...] = jnp.where(xix == 0, zrow, vr)
    acc_ref[...] = jnp.broadcast_to(b3_row, (th * w, c))
    for dy in range(2):
        s = pl.ds(dy * w, th * w)
        lhs = jnp.concatenate([br_ref[s], bc_ref[s], bl_ref[s]], axis=1)
        acc_ref[...] += jnp.dot(lhs, w3_ref[dy], preferred_element_type=F32)
    s = pl.ds(2 * w, th * w)
    lhs = jnp.concatenate([br_ref[s], bc_ref[s], bl_ref[s]], axis=1)
    return acc_ref[...] + jnp.dot(lhs, w3_ref[2], preferred_element_type=F32)


def _whole_body(H, W, C, x_ref, w1_ref, b1_ref, td_ref, w3_ref, b3_ref,
                r_ref, inner_ref, pool_ref, bc_ref, bl_ref, br_ref, acc_ref):
    lat = lax.dot_general(x_ref[0].astype(BF16), w1_ref[...].astype(BF16),
                          (((1,), (1,)), ((), ())),
                          preferred_element_type=F32) + b1_ref[...]
    if td_ref is not None:
        lat = lat + _up2x_rows(td_ref[0].astype(F32), H // 2, W // 2, C)
    inner = lat.astype(BF16)
    inner_ref[0] = inner

    zr = jnp.zeros((W, C), BF16)
    val = jnp.concatenate([zr, inner, zr], axis=0)

    res = _conv3x3_acc(val, w3_ref, b3_ref[...], bc_ref, bl_ref, br_ref,
                       acc_ref, H, W, C)
    r_ref[0] = res
    if pool_ref is not None:
        p = res.reshape(H // 2, 2, W // 2, 2, C)[:, 0, :, 0, :]
        pool_ref[0] = p.reshape((H // 2) * (W // 2), C)


def _level_whole(x_hwc, iw, ib, lw, lb, td=None, pool=False):
    N, HW, Cin = x_hwc.shape
    C = iw.shape[0]
    H = W = int(HW ** 0.5)
    assert H * W == HW
    w1 = iw.reshape(C, Cin)
    b1 = ib.reshape(1, C)
    w3 = jnp.transpose(lw, (2, 3, 1, 0)).reshape(3, 3 * C, C).astype(BF16)
    b3 = lb.reshape(1, C)

    in_specs = [
        pl.BlockSpec((1, HW, Cin), lambda n: (n, 0, 0)),
        pl.BlockSpec((C, Cin), lambda n: (0, 0)),
        pl.BlockSpec((1, C), lambda n: (0, 0)),
    ]
    args = [x_hwc, w1, b1]
    if td is not None:
        in_specs.append(pl.BlockSpec((1, HW // 4, C), lambda n: (n, 0, 0)))
        args.append(td)
    in_specs += [
        pl.BlockSpec((3, 3 * C, C), lambda n: (0, 0, 0)),
        pl.BlockSpec((1, C), lambda n: (0, 0)),
    ]
    args += [w3, b3]

    out_shape = [
        jax.ShapeDtypeStruct((N, HW, C), F32),
        jax.ShapeDtypeStruct((N, HW, C), BF16),
    ]
    out_specs = [
        pl.BlockSpec((1, HW, C), lambda n: (n, 0, 0)),
        pl.BlockSpec((1, HW, C), lambda n: (n, 0, 0)),
    ]
    if pool:
        out_shape.append(jax.ShapeDtypeStruct((N, HW // 4, C), F32))
        out_specs.append(pl.BlockSpec((1, HW // 4, C), lambda n: (n, 0, 0)))

    def kfn(*refs):
        it = iter(refs)
        x_ref = next(it); w1_ref = next(it); b1_ref = next(it)
        td_ref = next(it) if td is not None else None
        w3_ref = next(it); b3_ref = next(it)
        r_ref = next(it); inner_ref = next(it)
        pool_ref = next(it) if pool else None
        bc_ref = next(it); bl_ref = next(it); br_ref = next(it)
        acc_ref = next(it)
        _whole_body(H, W, C, x_ref, w1_ref, b1_ref, td_ref, w3_ref, b3_ref,
                    r_ref, inner_ref, pool_ref, bc_ref, bl_ref, br_ref, acc_ref)

    outs = pl.pallas_call(
        kfn,
        out_shape=out_shape,
        grid=(N,),
        in_specs=in_specs,
        out_specs=out_specs,
        scratch_shapes=[
            pltpu.VMEM(((H + 2) * W, C), BF16),
            pltpu.VMEM(((H + 2) * W, C), BF16),
            pltpu.VMEM(((H + 2) * W, C), BF16),
            pltpu.VMEM((HW, C), F32),
        ],
        compiler_params=pltpu.CompilerParams(
            dimension_semantics=("parallel",),
            vmem_limit_bytes=100 * 1024 * 1024,
        ),
    )(*args)
    return outs


def _make_l1_kernel(TH, W, C, NT):
    def kfn(xm_ref, xt_ref, xb_ref, w1_ref, b1_ref,
            tdm_ref, tdt_ref, tdb_ref, w3_ref, b3_ref,
            r_ref, inner_ref, bc_ref, bl_ref, br_ref, acc_ref):
        t = pl.program_id(1)
        w2 = W // 2

        def lat_dot(x2d):
            return lax.dot_general(x2d.astype(BF16), w1_ref[...].astype(BF16),
                                   (((1,), (1,)), ((), ())),
                                   preferred_element_type=F32) + b1_ref[...]

        lat = lat_dot(xm_ref[0])
        lat = lat + _up2x_rows(tdm_ref[0].astype(F32), TH // 2, w2, C)
        main = lat.astype(BF16)
        inner_ref[0] = main

        top = lat_dot(xt_ref[0])
        top = top + _up2x_cols(tdt_ref[0].astype(F32), w2, C)
        top = jnp.where(t > 0, top, 0.0).astype(BF16)

        bot = lat_dot(xb_ref[0])
        bot = bot + _up2x_cols(tdb_ref[0].astype(F32), w2, C)
        bot = jnp.where(t < NT - 1, bot, 0.0).astype(BF16)

        val = jnp.concatenate([top, main, bot], axis=0)
        res = _conv3x3_acc(val, w3_ref, b3_ref[...], bc_ref, bl_ref, br_ref,
                           acc_ref, TH, W, C)
        r_ref[0] = res
    return kfn


def _level1(x_hwc, iw, ib, lw, lb, td, TH=32):
    N, HW, Cin = x_hwc.shape
    C = iw.shape[0]
    H = W = int(HW ** 0.5)
    h2, w2 = H // 2, W // 2
    NT = H // TH
    w1 = iw.reshape(C, Cin)
    b1 = ib.reshape(1, C)
    w3 = jnp.transpose(lw, (2, 3, 1, 0)).reshape(3, 3 * C, C).astype(BF16)
    b3 = lb.reshape(1, C)

    TH2 = TH // 2
    in_specs = [
        pl.BlockSpec((1, TH * W, Cin), lambda n, t: (n, t, 0)),
        pl.BlockSpec((1, W, Cin), lambda n, t: (n, jnp.maximum(t * TH - 1, 0), 0)),
        pl.BlockSpec((1, W, Cin), lambda n, t: (n, jnp.minimum(t * TH + TH, H - 1), 0)),
        pl.BlockSpec((C, Cin), lambda n, t: (0, 0)),
        pl.BlockSpec((1, C), lambda n, t: (0, 0)),
        pl.BlockSpec((1, TH2 * w2, C), lambda n, t: (n, t, 0)),
        pl.BlockSpec((1, w2, C), lambda n, t: (n, jnp.maximum(t * TH2 - 1, 0), 0)),
        pl.BlockSpec((1, w2, C), lambda n, t: (n, jnp.minimum(t * TH2 + TH2, h2 - 1), 0)),
        pl.BlockSpec((3, 3 * C, C), lambda n, t: (0, 0, 0)),
        pl.BlockSpec((1, C), lambda n, t: (0, 0)),
    ]
    outs = pl.pallas_call(
        _make_l1_kernel(TH, W, C, NT),
        out_shape=[
            jax.ShapeDtypeStruct((N, HW, C), F32),
            jax.ShapeDtypeStruct((N, HW, C), BF16),
        ],
        grid=(N, NT),
        in_specs=in_specs,
        out_specs=[
            pl.BlockSpec((1, TH * W, C), lambda n, t: (n, t, 0)),
            pl.BlockSpec((1, TH * W, C), lambda n, t: (n, t, 0)),
        ],
        scratch_shapes=[
            pltpu.VMEM(((TH + 2) * W, C), BF16),
            pltpu.VMEM(((TH + 2) * W, C), BF16),
            pltpu.VMEM(((TH + 2) * W, C), BF16),
            pltpu.VMEM((TH * W, C), F32),
        ],
        compiler_params=pltpu.CompilerParams(
            dimension_semantics=("parallel", "arbitrary"),
            vmem_limit_bytes=100 * 1024 * 1024,
        ),
    )(x_hwc, x_hwc, x_hwc, w1, b1, td, td, td, w3, b3)
    return outs


def _make_l0_kernel(TH, W, C, NT):
    def kfn(xm_ref, xt_ref, xb_ref, w1_ref, b1_ref,
            tdm_ref, tdt_ref, tdb_ref, w3_ref, b3_ref,
            r_ref, bc_ref, bl_ref, br_ref, acc_ref):
        t = pl.program_id(1)
        w2 = W // 2
        cin = xm_ref.shape[1]

        def lat_dot(x2d):
            return lax.dot_general(x2d.astype(BF16), w1_ref[...].astype(BF16),
                                   (((0,), (1,)), ((), ())),
                                   preferred_element_type=F32) + b1_ref[...]

        lat = lat_dot(xm_ref[0].reshape(cin, TH * W))
        lat = lat + _up2x_rows(tdm_ref[0].astype(F32), TH // 2, w2, C)
        main = lat.astype(BF16)

        top = lat_dot(xt_ref[0, :, 7, :])
        top = top + _up2x_cols(tdt_ref[0].astype(F32), w2, C)
        top = jnp.where(t > 0, top, 0.0).astype(BF16)

        bot = lat_dot(xb_ref[0, :, 0, :])
        bot = bot + _up2x_cols(tdb_ref[0].astype(F32), w2, C)
        bot = jnp.where(t < NT - 1, bot, 0.0).astype(BF16)

        val = jnp.concatenate([top, main, bot], axis=0)
        res = _conv3x3_acc(val, w3_ref, b3_ref[...], bc_ref, bl_ref, br_ref,
                           acc_ref, TH, W, C)
        r = jnp.transpose(res)
        r_ref[0] = r.reshape(C, TH, W)
    return kfn


def _level0(x_nchw, iw, ib, lw, lb, td, TH=32):
    N, Cin, H, W = x_nchw.shape
    C = iw.shape[0]
    h2, w2 = H // 2, W // 2
    NT = H // TH
    w1 = iw.reshape(C, Cin)
    b1 = ib.reshape(1, C)
    w3 = jnp.transpose(lw, (2, 3, 1, 0)).reshape(3, 3 * C, C).astype(BF16)
    b3 = lb.reshape(1, C)

    TH2 = TH // 2
    in_specs = [
        pl.BlockSpec((1, Cin, TH, W), lambda n, t: (n, 0, t, 0)),
        pl.BlockSpec((1, Cin, 8, W),
                     lambda n, t: (n, 0, jnp.maximum(t * (TH // 8) - 1, 0), 0)),
        pl.BlockSpec((1, Cin, 8, W),
                     lambda n, t: (n, 0, jnp.minimum((t + 1) * (TH // 8), H // 8 - 1), 0)),
        pl.BlockSpec((C, Cin), lambda n, t: (0, 0)),
        pl.BlockSpec((1, C), lambda n, t: (0, 0)),
        pl.BlockSpec((1, TH2 * w2, C), lambda n, t: (n, t, 0)),
        pl.BlockSpec((1, w2, C), lambda n, t: (n, jnp.maximum(t * TH2 - 1, 0), 0)),
        pl.BlockSpec((1, w2, C), lambda n, t: (n, jnp.minimum(t * TH2 + TH2, h2 - 1), 0)),
        pl.BlockSpec((3, 3 * C, C), lambda n, t: (0, 0, 0)),
        pl.BlockSpec((1, C), lambda n, t: (0, 0)),
    ]
    out = pl.pallas_call(
        _make_l0_kernel(TH, W, C, NT),
        out_shape=jax.ShapeDtypeStruct((N, C, H, W), F32),
        grid=(N, NT),
        in_specs=in_specs,
        out_specs=pl.BlockSpec((1, C, TH, W), lambda n, t: (n, 0, t, 0)),
        scratch_shapes=[
            pltpu.VMEM(((TH + 2) * W, C), BF16),
            pltpu.VMEM(((TH + 2) * W, C), BF16),
            pltpu.VMEM(((TH + 2) * W, C), BF16),
            pltpu.VMEM((TH * W, C), F32),
        ],
        compiler_params=pltpu.CompilerParams(
            dimension_semantics=("parallel", "arbitrary"),
            vmem_limit_bytes=100 * 1024 * 1024,
        ),
    )(x_nchw, x_nchw, x_nchw, w1, b1, td, td, td, w3, b3)
    return out


def _to_nchw(r_hwc, N, C, H, W):
    return jnp.transpose(r_hwc, (0, 2, 1)).reshape(N, C, H, W)


def kernel(x0, x1, x2, iw0, ib0, lw0, lb0, iw1, ib1, lw1, lb1, iw2, ib2, lw2, lb2):
    N = x0.shape[0]
    C = iw0.shape[0]
    H0, H1, H2 = x0.shape[2], x1.shape[2], x2.shape[2]

    xh1 = jnp.transpose(x1, (0, 2, 3, 1)).reshape(N, H1 * H1, x1.shape[1])
    xh2 = jnp.transpose(x2, (0, 2, 3, 1)).reshape(N, H2 * H2, x2.shape[1])

    r2f, inner2, poolf = _level_whole(xh2, iw2, ib2, lw2, lb2, pool=True)

    r1f, inner1 = _level1(xh1, iw1, ib1, lw1, lb1, td=inner2)

    r0f = _level0(x0, iw0, ib0, lw0, lb0, td=inner1)

    r0 = r0f
    r1 = _to_nchw(r1f, N, C, H1, H1)
    r2 = _to_nchw(r2f, N, C, H2, H2)
    pool = _to_nchw(poolf, N, C, H2 // 2, H2 // 2)
    return (r0, r1, r2, pool)
